# baseline (speedup 1.0000x reference)
import functools

import jax
import jax.numpy as jnp
from jax import lax
from jax.experimental import pallas as pl
from jax.experimental.pallas import tpu as pltpu
from jax.experimental.pallas import tpu_sc as plsc

BATCH = 16384
OUT_DIM = 64
_WIDE = 2 * OUT_DIM

_NUM_CORES = 2
_NUM_SUBCORES = 16
_NUM_WORKERS = _NUM_CORES * _NUM_SUBCORES
_B_PER_W = BATCH // _NUM_WORKERS
_CHUNK = 128
_N_CHUNKS = _B_PER_W // _CHUNK
_LANES = 16

_mesh = plsc.VectorSubcoreMesh(core_axis_name="c", subcore_axis_name="s")


@functools.partial(
    pl.kernel,
    out_type=jax.ShapeDtypeStruct((BATCH, _WIDE), jnp.float32),
    mesh=_mesh,
    scratch_types=(
        [pltpu.VMEM((_B_PER_W,), jnp.int32)]
        + [pltpu.VMEM((_CHUNK,), jnp.int32) for _ in range(_N_CHUNKS)]
        + [pltpu.VMEM((_CHUNK, _WIDE), jnp.float32) for _ in range(_N_CHUNKS)]
        + [pltpu.SemaphoreType.DMA]
    ),
)
def _embed_gather(labels_hbm, table_hbm, out_hbm, lab_v, *scratch):
    idx_vs = scratch[:_N_CHUNKS]
    wide_vs = scratch[_N_CHUNKS:2 * _N_CHUNKS]
    sem = scratch[2 * _N_CHUNKS]

    wid = lax.axis_index("s") * _NUM_CORES + lax.axis_index("c")
    base = wid * _B_PER_W
    pltpu.sync_copy(labels_hbm.at[pl.ds(base, _B_PER_W)], lab_v)

    for k in range(_N_CHUNKS):
        for g in range(_CHUNK // _LANES):
            lv = lab_v[pl.ds(k * _CHUNK + g * _LANES, _LANES)]
            idx_vs[k][pl.ds(g * _LANES, _LANES)] = lv >> 1
    copies = [
        pltpu.async_copy(table_hbm.at[idx_vs[k]], wide_vs[k], sem)
        for k in range(_N_CHUNKS)
    ]
    for k in range(_N_CHUNKS):
        copies[k].wait()
        pltpu.sync_copy(
            wide_vs[k], out_hbm.at[pl.ds(base + k * _CHUNK, _CHUNK)]
        )


def kernel(labels, table, training=0):
    del training
    labels = labels.astype(jnp.int32)
    table_wide = table.reshape(table.shape[0] // 2, _WIDE)
    wide = _embed_gather(labels, table_wide)
    right = (labels & 1).astype(jnp.bool_)
    return jnp.where(right[:, None], wide[:, OUT_DIM:], wide[:, :OUT_DIM])
